# baseline (speedup 1.0000x reference)
import functools

import jax
import jax.numpy as jnp
from jax import lax
from jax.experimental import pallas as pl
from jax.experimental.pallas import tpu as pltpu
from jax.experimental.pallas import tpu_sc as plsc

N_CORES = 2
N_SUB = 16
NW = N_CORES * N_SUB
K = 16
CH = K * 128


def _sc_pass(src3, dst3, u, zeros, npad, with_gather):
    rows = dst3.shape[0]
    m = rows // (NW * K)
    sl = npad // N_SUB

    def _drain(zeros_hbm, dst2d, sem, n):
        for b in range(n):
            pltpu.make_async_copy(zeros_hbm.at[pl.ds(0, 128)],
                                  dst2d.at[b % dst2d.shape[0]], sem).wait()

    def _wait_lin(hbm, dst, sem):
        pltpu.make_async_copy(hbm.at[pl.ds(0, K)], dst, sem).wait()

    def gather_body(src_hbm, dst_hbm, u_hbm, zeros_hbm, out_hbm,
                    sidxA, sidxB, didxA, didxB, valsA, valsB,
                    tmp, acc, lsem, gsem, ssem0, ssem1):
        c = lax.axis_index("c")
        s = lax.axis_index("s")
        wid = c * N_SUB + s
        base_n = s * sl
        pltpu.sync_copy(zeros_hbm.at[pl.ds(base_n, sl)], tmp)
        pltpu.sync_copy(tmp, acc.at[pl.ds(base_n, sl)])
        for b in range(K):
            for v in range(8):
                valsB[b, pl.ds(16 * v, 16)] = jnp.zeros((16,), jnp.float32)
                didxB[b, pl.ds(16 * v, 16)] = jnp.zeros((16,), jnp.int32)
        plsc.subcore_barrier()
        for b in range(K):
            pltpu.async_copy(valsB.at[b], acc.at[didxB.at[b]], ssem1, add=True)
        row0 = wid * (m * K)
        pltpu.async_copy(src_hbm.at[pl.ds(row0, K)], sidxA, lsem)
        pltpu.async_copy(dst_hbm.at[pl.ds(row0, K)], didxA, lsem)

        def pair(j, carry):
            rb1 = row0 + (2 * j + 1) * K
            rb2 = jnp.minimum(row0 + (2 * j + 2) * K, rows - K)
            _wait_lin(src_hbm, sidxA, lsem)
            _wait_lin(dst_hbm, didxA, lsem)
            for b in range(K):
                pltpu.async_copy(u_hbm.at[sidxA.at[b]], valsA.at[b], gsem)
            _drain(zeros_hbm, valsB, ssem1, K)
            pltpu.async_copy(src_hbm.at[pl.ds(rb1, K)], sidxB, lsem)
            pltpu.async_copy(dst_hbm.at[pl.ds(rb1, K)], didxB, lsem)
            _drain(zeros_hbm, valsA, gsem, K)
            for b in range(K):
                pltpu.async_copy(valsA.at[b], acc.at[didxA.at[b]], ssem0,
                                 add=True)
            _wait_lin(src_hbm, sidxB, lsem)
            _wait_lin(dst_hbm, didxB, lsem)
            for b in range(K):
                pltpu.async_copy(u_hbm.at[sidxB.at[b]], valsB.at[b], gsem)
            _drain(zeros_hbm, valsA, ssem0, K)
            pltpu.async_copy(src_hbm.at[pl.ds(rb2, K)], sidxA, lsem)
            pltpu.async_copy(dst_hbm.at[pl.ds(rb2, K)], didxA, lsem)
            _drain(zeros_hbm, valsB, gsem, K)
            for b in range(K):
                pltpu.async_copy(valsB.at[b], acc.at[didxB.at[b]], ssem1,
                                 add=True)
            return carry

        lax.fori_loop(0, m // 2, pair, 0)
        _drain(zeros_hbm, valsB, ssem1, K)
        _wait_lin(src_hbm, sidxA, lsem)
        _wait_lin(dst_hbm, didxA, lsem)
        plsc.subcore_barrier()
        pltpu.sync_copy(acc.at[pl.ds(base_n, sl)], tmp)
        pltpu.sync_copy(tmp, out_hbm.at[c, pl.ds(base_n, sl)])

    def deg_body(src_hbm, dst_hbm, u_hbm, zeros_hbm, out_hbm,
                 didxA, didxB, onesv, zerov, tmp, acc, lsem, ssem0, ssem1):
        c = lax.axis_index("c")
        s = lax.axis_index("s")
        wid = c * N_SUB + s
        base_n = s * sl
        pltpu.sync_copy(zeros_hbm.at[pl.ds(base_n, sl)], tmp)
        pltpu.sync_copy(tmp, acc.at[pl.ds(base_n, sl)])
        for v in range(8):
            onesv[pl.ds(16 * v, 16)] = jnp.ones((16,), jnp.float32)
        for b in range(K):
            for v in range(8):
                zerov[b, pl.ds(16 * v, 16)] = jnp.zeros((16,), jnp.float32)
                didxB[b, pl.ds(16 * v, 16)] = jnp.zeros((16,), jnp.int32)
        plsc.subcore_barrier()
        for b in range(K):
            pltpu.async_copy(zerov.at[b], acc.at[didxB.at[b]], ssem1, add=True)
        row0 = wid * (m * K)
        pltpu.async_copy(dst_hbm.at[pl.ds(row0, K)], didxA, lsem)

        def pair(j, carry):
            rb1 = row0 + (2 * j + 1) * K
            rb2 = jnp.minimum(row0 + (2 * j + 2) * K, rows - K)
            _wait_lin(dst_hbm, didxA, lsem)
            _drain(zeros_hbm, zerov, ssem1, K)
            for b in range(K):
                pltpu.async_copy(onesv, acc.at[didxA.at[b]], ssem0, add=True)
            pltpu.async_copy(dst_hbm.at[pl.ds(rb1, K)], didxB, lsem)
            _wait_lin(dst_hbm, didxB, lsem)
            for b in range(K):
                pltpu.async_copy(onesv, acc.at[didxB.at[b]], ssem1, add=True)
            _drain(zeros_hbm, zerov, ssem0, K)
            pltpu.async_copy(dst_hbm.at[pl.ds(rb2, K)], didxA, lsem)
            return carry

        lax.fori_loop(0, m // 2, pair, 0)
        _drain(zeros_hbm, zerov, ssem1, K)
        _wait_lin(dst_hbm, didxA, lsem)
        plsc.subcore_barrier()
        pltpu.sync_copy(acc.at[pl.ds(base_n, sl)], tmp)
        pltpu.sync_copy(tmp, out_hbm.at[c, pl.ds(base_n, sl)])

    mesh = plsc.VectorSubcoreMesh(core_axis_name="c", subcore_axis_name="s")
    if with_gather:
        scratch = [
            pltpu.VMEM((K, 128), jnp.int32),
            pltpu.VMEM((K, 128), jnp.int32),
            pltpu.VMEM((K, 128), jnp.int32),
            pltpu.VMEM((K, 128), jnp.int32),
            pltpu.VMEM((K, 128), jnp.float32),
            pltpu.VMEM((K, 128), jnp.float32),
            pltpu.VMEM((sl,), jnp.float32),
            pltpu.VMEM_SHARED((npad,), jnp.float32),
            pltpu.SemaphoreType.DMA,
            pltpu.SemaphoreType.DMA,
            pltpu.SemaphoreType.DMA,
            pltpu.SemaphoreType.DMA,
        ]
        body = gather_body
    else:
        scratch = [
            pltpu.VMEM((K, 128), jnp.int32),
            pltpu.VMEM((K, 128), jnp.int32),
            pltpu.VMEM((128,), jnp.float32),
            pltpu.VMEM((K, 128), jnp.float32),
            pltpu.VMEM((sl,), jnp.float32),
            pltpu.VMEM_SHARED((npad,), jnp.float32),
            pltpu.SemaphoreType.DMA,
            pltpu.SemaphoreType.DMA,
            pltpu.SemaphoreType.DMA,
        ]
        body = deg_body
    f = pl.kernel(
        body,
        out_type=jax.ShapeDtypeStruct((N_CORES, npad), jnp.float32),
        mesh=mesh,
        scratch_types=scratch,
        name="gcn_edge_pass" + ("_gather" if with_gather else "_deg"),
    )
    return f(src3, dst3, u, zeros)


def _tc_prep(degp, xt, scal, r):
    def body(degp_ref, xt_ref, sc_ref, dinv_ref, u1_ref):
        deg = jnp.sum(degp_ref[...], axis=0) + 1.0
        dinv = lax.rsqrt(deg)
        h = (xt_ref[0] * sc_ref[0] + xt_ref[1] * sc_ref[1]
             + xt_ref[2] * sc_ref[2])
        dinv_ref[...] = dinv
        u1_ref[...] = dinv * h

    return pl.pallas_call(
        body,
        out_shape=(jax.ShapeDtypeStruct((r, 128), jnp.float32),
                   jax.ShapeDtypeStruct((r, 128), jnp.float32)),
        in_specs=[pl.BlockSpec(memory_space=pltpu.VMEM),
                  pl.BlockSpec(memory_space=pltpu.VMEM),
                  pl.BlockSpec(memory_space=pltpu.SMEM)],
        out_specs=(pl.BlockSpec(memory_space=pltpu.VMEM),
                   pl.BlockSpec(memory_space=pltpu.VMEM)),
    )(degp, xt, scal)


def _tc_mid(sp, u1, dinv, scal, r):
    def body(sp_ref, u1_ref, dinv_ref, sc_ref, u2_ref):
        dinv = dinv_ref[...]
        agg = dinv * (jnp.sum(sp_ref[...], axis=0) + u1_ref[...]) + sc_ref[3]
        h1 = jnp.maximum(agg, 0.0)
        u2_ref[...] = dinv * (h1 * sc_ref[4])

    return pl.pallas_call(
        body,
        out_shape=jax.ShapeDtypeStruct((r, 128), jnp.float32),
        in_specs=[pl.BlockSpec(memory_space=pltpu.VMEM),
                  pl.BlockSpec(memory_space=pltpu.VMEM),
                  pl.BlockSpec(memory_space=pltpu.VMEM),
                  pl.BlockSpec(memory_space=pltpu.SMEM)],
        out_specs=pl.BlockSpec(memory_space=pltpu.VMEM),
    )(sp, u1, dinv, scal)


def _tc_final(sp, u2, dinv, coxp, scal, r):
    def body(sp_ref, u2_ref, dinv_ref, coxp_ref, sc_ref, gnn_ref, risk_ref):
        gnn = (dinv_ref[...] * (jnp.sum(sp_ref[...], axis=0) + u2_ref[...])
               + sc_ref[5])
        gnn_ref[...] = gnn
        risk_ref[0, 0] = jnp.sum(gnn * coxp_ref[...]) + sc_ref[6]

    return pl.pallas_call(
        body,
        out_shape=(jax.ShapeDtypeStruct((r, 128), jnp.float32),
                   jax.ShapeDtypeStruct((1, 1), jnp.float32)),
        in_specs=[pl.BlockSpec(memory_space=pltpu.VMEM),
                  pl.BlockSpec(memory_space=pltpu.VMEM),
                  pl.BlockSpec(memory_space=pltpu.VMEM),
                  pl.BlockSpec(memory_space=pltpu.VMEM),
                  pl.BlockSpec(memory_space=pltpu.SMEM)],
        out_specs=(pl.BlockSpec(memory_space=pltpu.VMEM),
                   pl.BlockSpec(memory_space=pltpu.SMEM)),
    )(sp, u2, dinv, coxp, scal)


def kernel(x, edge_index, W1, b1, W2, b2, cox_W, cox_b):
    n = x.shape[0]
    e = edge_index.shape[1]
    npad = ((n + 16 * 128 - 1) // (16 * 128)) * (16 * 128)
    r = npad // 128
    m = -(-e // (NW * CH))
    m += m % 2
    epad = NW * CH * m

    ei = edge_index.astype(jnp.int32)
    pad_ids = n + (jnp.arange(epad - e, dtype=jnp.int32) % 64)
    src3 = jnp.concatenate([ei[0], pad_ids]).reshape(epad // 128, 128)
    dst3 = jnp.concatenate([ei[1], pad_ids]).reshape(epad // 128, 128)
    zeros = jnp.zeros((npad,), jnp.float32)
    xt = jnp.pad(x.T, ((0, 0), (0, npad - n))).reshape(3, r, 128)
    coxp = jnp.pad(cox_W[0], (0, npad - n)).reshape(r, 128)
    scal = jnp.concatenate([W1[:, 0], b1, W2[0], b2, cox_b,
                            jnp.zeros((1,), jnp.float32)])

    degp = _sc_pass(src3, dst3, zeros, zeros, npad, with_gather=False)
    dinv, u1 = _tc_prep(degp.reshape(N_CORES, r, 128), xt, scal, r)
    s1 = _sc_pass(src3, dst3, u1.reshape(npad), zeros, npad, with_gather=True)
    u2 = _tc_mid(s1.reshape(N_CORES, r, 128), u1, dinv, scal, r)
    s2 = _sc_pass(src3, dst3, u2.reshape(npad), zeros, npad, with_gather=True)
    gnn, risk = _tc_final(s2.reshape(N_CORES, r, 128), u2, dinv, coxp, scal, r)

    gnn_t = gnn.reshape(npad)[:n].reshape(1, n)
    return risk, gnn_t


# speedup vs baseline: 2.0853x; 2.0853x over previous
import functools

import jax
import jax.numpy as jnp
from jax import lax
from jax.experimental import pallas as pl
from jax.experimental.pallas import tpu as pltpu
from jax.experimental.pallas import tpu_sc as plsc

N_CORES = 2
N_SUB = 16
NW = N_CORES * N_SUB
K = 16
CH = K * 128


def _sc_pass(src3, dst3, u, zeros, npad, with_gather):
    rows = dst3.shape[0]
    m = rows // (NW * K)
    sl = npad // N_SUB

    def _drain(zeros_hbm, dst2d, sem, n):
        for b in range(n):
            pltpu.make_async_copy(zeros_hbm.at[pl.ds(0, 128)],
                                  dst2d.at[b % dst2d.shape[0]], sem).wait()

    def _wait_lin(hbm, dst, sem):
        pltpu.make_async_copy(hbm.at[pl.ds(0, K)], dst, sem).wait()

    def _scat(didx, vals, acc):
        for b in range(K):
            for v in range(8):
                ix = didx[b, pl.ds(16 * v, 16)]
                vv = vals[b, pl.ds(16 * v, 16)]
                plsc.addupdate_scatter(acc, [ix], vv)

    def gather_body(src_hbm, dst_hbm, u_hbm, zeros_hbm, out_hbm,
                    sidxA, sidxB, didxA, didxB, valsA, valsB,
                    tmp, acc, u_sh, lsem, gsem):
        c = lax.axis_index("c")
        s = lax.axis_index("s")
        wid = c * N_SUB + s
        base_n = s * sl
        pltpu.sync_copy(zeros_hbm, acc)
        pltpu.sync_copy(u_hbm.at[pl.ds(base_n, sl)], tmp)
        pltpu.sync_copy(tmp, u_sh.at[pl.ds(base_n, sl)])
        plsc.subcore_barrier()
        row0 = wid * (m * K)
        pltpu.async_copy(src_hbm.at[pl.ds(row0, K)], sidxA, lsem)
        pltpu.async_copy(dst_hbm.at[pl.ds(row0, K)], didxA, lsem)

        def pair(j, carry):
            rb1 = row0 + (2 * j + 1) * K
            rb2 = jnp.minimum(row0 + (2 * j + 2) * K, rows - K)
            _wait_lin(src_hbm, sidxA, lsem)
            _wait_lin(dst_hbm, didxA, lsem)
            for b in range(K):
                pltpu.async_copy(u_sh.at[sidxA.at[b]], valsA.at[b], gsem)
            pltpu.async_copy(src_hbm.at[pl.ds(rb1, K)], sidxB, lsem)
            pltpu.async_copy(dst_hbm.at[pl.ds(rb1, K)], didxB, lsem)
            _drain(zeros_hbm, valsA, gsem, K)
            _wait_lin(src_hbm, sidxB, lsem)
            _wait_lin(dst_hbm, didxB, lsem)
            for b in range(K):
                pltpu.async_copy(u_sh.at[sidxB.at[b]], valsB.at[b], gsem)
            _scat(didxA, valsA, acc)
            pltpu.async_copy(src_hbm.at[pl.ds(rb2, K)], sidxA, lsem)
            pltpu.async_copy(dst_hbm.at[pl.ds(rb2, K)], didxA, lsem)
            _drain(zeros_hbm, valsB, gsem, K)
            _scat(didxB, valsB, acc)
            return carry

        lax.fori_loop(0, m // 2, pair, 0)
        _wait_lin(src_hbm, sidxA, lsem)
        _wait_lin(dst_hbm, didxA, lsem)
        pltpu.sync_copy(acc, out_hbm.at[wid])

    def _scat1(didx, acc):
        for b in range(K):
            for v in range(8):
                ix = didx[b, pl.ds(16 * v, 16)]
                plsc.addupdate_scatter(acc, [ix],
                                       jnp.ones((16,), jnp.float32))

    def deg_body(src_hbm, dst_hbm, u_hbm, zeros_hbm, out_hbm,
                 didxA, didxB, acc, lsem):
        c = lax.axis_index("c")
        s = lax.axis_index("s")
        wid = c * N_SUB + s
        pltpu.sync_copy(zeros_hbm, acc)
        row0 = wid * (m * K)
        pltpu.async_copy(dst_hbm.at[pl.ds(row0, K)], didxA, lsem)

        def pair(j, carry):
            rb1 = row0 + (2 * j + 1) * K
            rb2 = jnp.minimum(row0 + (2 * j + 2) * K, rows - K)
            _wait_lin(dst_hbm, didxA, lsem)
            pltpu.async_copy(dst_hbm.at[pl.ds(rb1, K)], didxB, lsem)
            _scat1(didxA, acc)
            _wait_lin(dst_hbm, didxB, lsem)
            pltpu.async_copy(dst_hbm.at[pl.ds(rb2, K)], didxA, lsem)
            _scat1(didxB, acc)
            return carry

        lax.fori_loop(0, m // 2, pair, 0)
        _wait_lin(dst_hbm, didxA, lsem)
        pltpu.sync_copy(acc, out_hbm.at[wid])

    mesh = plsc.VectorSubcoreMesh(core_axis_name="c", subcore_axis_name="s")
    if with_gather:
        scratch = [
            pltpu.VMEM((K, 128), jnp.int32),
            pltpu.VMEM((K, 128), jnp.int32),
            pltpu.VMEM((K, 128), jnp.int32),
            pltpu.VMEM((K, 128), jnp.int32),
            pltpu.VMEM((K, 128), jnp.float32),
            pltpu.VMEM((K, 128), jnp.float32),
            pltpu.VMEM((sl,), jnp.float32),
            pltpu.VMEM((npad,), jnp.float32),
            pltpu.VMEM_SHARED((npad,), jnp.float32),
            pltpu.SemaphoreType.DMA,
            pltpu.SemaphoreType.DMA,
        ]
        body = gather_body
    else:
        scratch = [
            pltpu.VMEM((K, 128), jnp.int32),
            pltpu.VMEM((K, 128), jnp.int32),
            pltpu.VMEM((npad,), jnp.float32),
            pltpu.SemaphoreType.DMA,
        ]
        body = deg_body
    f = pl.kernel(
        body,
        out_type=jax.ShapeDtypeStruct((NW, npad), jnp.float32),
        mesh=mesh,
        scratch_types=scratch,
        name="gcn_edge_pass" + ("_gather" if with_gather else "_deg"),
        compiler_params=pltpu.CompilerParams(needs_layout_passes=False),
    )
    return f(src3, dst3, u, zeros)


def _tc_prep(degp, xt, scal, r):
    def body(degp_ref, xt_ref, sc_ref, dinv_ref, u1_ref):
        deg = jnp.sum(degp_ref[...], axis=0) + 1.0
        dinv = lax.rsqrt(deg)
        h = (xt_ref[0] * sc_ref[0] + xt_ref[1] * sc_ref[1]
             + xt_ref[2] * sc_ref[2])
        dinv_ref[...] = dinv
        u1_ref[...] = dinv * h

    return pl.pallas_call(
        body,
        out_shape=(jax.ShapeDtypeStruct((r, 128), jnp.float32),
                   jax.ShapeDtypeStruct((r, 128), jnp.float32)),
        in_specs=[pl.BlockSpec(memory_space=pltpu.VMEM),
                  pl.BlockSpec(memory_space=pltpu.VMEM),
                  pl.BlockSpec(memory_space=pltpu.SMEM)],
        out_specs=(pl.BlockSpec(memory_space=pltpu.VMEM),
                   pl.BlockSpec(memory_space=pltpu.VMEM)),
    )(degp, xt, scal)


def _tc_mid(sp, u1, dinv, scal, r):
    def body(sp_ref, u1_ref, dinv_ref, sc_ref, u2_ref):
        dinv = dinv_ref[...]
        agg = dinv * (jnp.sum(sp_ref[...], axis=0) + u1_ref[...]) + sc_ref[3]
        h1 = jnp.maximum(agg, 0.0)
        u2_ref[...] = dinv * (h1 * sc_ref[4])

    return pl.pallas_call(
        body,
        out_shape=jax.ShapeDtypeStruct((r, 128), jnp.float32),
        in_specs=[pl.BlockSpec(memory_space=pltpu.VMEM),
                  pl.BlockSpec(memory_space=pltpu.VMEM),
                  pl.BlockSpec(memory_space=pltpu.VMEM),
                  pl.BlockSpec(memory_space=pltpu.SMEM)],
        out_specs=pl.BlockSpec(memory_space=pltpu.VMEM),
    )(sp, u1, dinv, scal)


def _tc_final(sp, u2, dinv, coxp, scal, r):
    def body(sp_ref, u2_ref, dinv_ref, coxp_ref, sc_ref, gnn_ref, risk_ref):
        gnn = (dinv_ref[...] * (jnp.sum(sp_ref[...], axis=0) + u2_ref[...])
               + sc_ref[5])
        gnn_ref[...] = gnn
        risk_ref[0, 0] = jnp.sum(gnn * coxp_ref[...]) + sc_ref[6]

    return pl.pallas_call(
        body,
        out_shape=(jax.ShapeDtypeStruct((r, 128), jnp.float32),
                   jax.ShapeDtypeStruct((1, 1), jnp.float32)),
        in_specs=[pl.BlockSpec(memory_space=pltpu.VMEM),
                  pl.BlockSpec(memory_space=pltpu.VMEM),
                  pl.BlockSpec(memory_space=pltpu.VMEM),
                  pl.BlockSpec(memory_space=pltpu.VMEM),
                  pl.BlockSpec(memory_space=pltpu.SMEM)],
        out_specs=(pl.BlockSpec(memory_space=pltpu.VMEM),
                   pl.BlockSpec(memory_space=pltpu.SMEM)),
    )(sp, u2, dinv, coxp, scal)


def kernel(x, edge_index, W1, b1, W2, b2, cox_W, cox_b):
    n = x.shape[0]
    e = edge_index.shape[1]
    npad = ((n + 16 * 128 - 1) // (16 * 128)) * (16 * 128)
    r = npad // 128
    m = -(-e // (NW * CH))
    m += m % 2
    epad = NW * CH * m

    ei = edge_index.astype(jnp.int32)
    pad_ids = n + (jnp.arange(epad - e, dtype=jnp.int32) % 64)
    src3 = jnp.concatenate([ei[0], pad_ids]).reshape(epad // 128, 128)
    dst3 = jnp.concatenate([ei[1], pad_ids]).reshape(epad // 128, 128)
    zeros = jnp.zeros((npad,), jnp.float32)
    xt = jnp.pad(x.T, ((0, 0), (0, npad - n))).reshape(3, r, 128)
    coxp = jnp.pad(cox_W[0], (0, npad - n)).reshape(r, 128)
    scal = jnp.concatenate([W1[:, 0], b1, W2[0], b2, cox_b,
                            jnp.zeros((1,), jnp.float32)])

    degp = _sc_pass(src3, dst3, zeros, zeros, npad, with_gather=False)
    dinv, u1 = _tc_prep(degp.reshape(NW, r, 128), xt, scal, r)
    s1 = _sc_pass(src3, dst3, u1.reshape(npad), zeros, npad, with_gather=True)
    u2 = _tc_mid(s1.reshape(NW, r, 128), u1, dinv, scal, r)
    s2 = _sc_pass(src3, dst3, u2.reshape(npad), zeros, npad, with_gather=True)
    gnn, risk = _tc_final(s2.reshape(NW, r, 128), u2, dinv, coxp, scal, r)

    gnn_t = gnn.reshape(npad)[:n].reshape(1, n)
    return risk, gnn_t
